# Initial kernel scaffold; baseline (speedup 1.0000x reference)
import jax
import jax.numpy as jnp
from jax.experimental import pallas as pl


def kernel(queries, keys, W):
    raise NotImplementedError("write your pallas kernel here")


# speedup vs baseline: 3.2579x; 3.2579x over previous
import functools
import jax
import jax.numpy as jnp
from jax.experimental import pallas as pl
from jax.experimental.pallas import tpu as pltpu

TOP_K = 16
Q, K, D = 1024, 100000, 128
KPAD = 102400
GRP = 32
NGRP = KPAD // GRP
QBLK = 512
KBLK = 4096
NKSTEP = KPAD // KBLK
GCOLS = KBLK // GRP
NSEL = 16
BIG = float(3.0e38)


NCHUNK = NGRP // GCOLS
MERGEW = 512


def _k1_body(q_ref, k_ref, s_out, gc_out, q2_out, gm_ref, mv_ref, mi_ref):
    ki = pl.program_id(1)
    qb = q_ref[...]
    kb = k_ref[...]
    cross = jax.lax.dot_general(
        (-2.0 * qb).astype(jnp.bfloat16), kb.astype(jnp.bfloat16),
        (((1,), (1,)), ((), ())),
        preferred_element_type=jnp.float32)
    k2b = jax.lax.dot_general(
        jnp.ones((QBLK, D), jnp.float32), kb * kb,
        (((1,), (1,)), ((), ())),
        precision=jax.lax.Precision.HIGHEST,
        preferred_element_type=jnp.float32)
    s_out[...] = cross + k2b

    def _gm_iter(i, m):
        return jnp.minimum(m, s_out[:, pl.ds(i * GCOLS, GCOLS)])
    gm_ref[:, pl.ds(ki * GCOLS, GCOLS)] = jax.lax.fori_loop(
        1, GRP, _gm_iter, s_out[:, 0:GCOLS])

    q2_out[...] = jnp.sum(qb * qb, axis=1, keepdims=True)
    gc_out[...] = jnp.zeros((QBLK, NSEL), jnp.int32)

    @pl.when(ki == NKSTEP - 1)
    def _():
        lane16 = jax.lax.broadcasted_iota(jnp.int32, (QBLK, NSEL), 1)
        ciota = jax.lax.broadcasted_iota(jnp.int32, (QBLK, GCOLS), 1)

        all_v, all_i = [], []
        for j in range(NCHUNK):
            gids = ciota + j * GCOLS
            sl = slice(j * GCOLS, (j + 1) * GCOLS)

            def _sel(t, carry, sl=sl, gids=gids):
                vcol, icol = carry
                vals = gm_ref[:, sl]
                m = jnp.min(vals, axis=1, keepdims=True)
                am = jnp.min(jnp.where(vals == m, gids, jnp.int32(2**30)),
                             axis=1, keepdims=True)
                gm_ref[:, sl] = jnp.where(gids == am, BIG, vals)
                return (jnp.where(lane16 == t, m, vcol),
                        jnp.where(lane16 == t, am, icol))

            vcol, icol = jax.lax.fori_loop(
                0, NSEL, _sel,
                (jnp.full((QBLK, NSEL), BIG, jnp.float32),
                 jnp.zeros((QBLK, NSEL), jnp.int32)))
            all_v.append(vcol)
            all_i.append(icol)
        padw = MERGEW - NCHUNK * NSEL
        mv_ref[...] = jnp.concatenate(
            all_v + [jnp.full((QBLK, padw), BIG, jnp.float32)], axis=1)
        mi_ref[...] = jnp.concatenate(
            all_i + [jnp.zeros((QBLK, padw), jnp.int32)], axis=1)

        piota = jax.lax.broadcasted_iota(jnp.int32, (QBLK, MERGEW), 1)

        def _merge(t, gc):
            vals = mv_ref[...]
            m = jnp.min(vals, axis=1, keepdims=True)
            ap = jnp.min(jnp.where(vals == m, piota, jnp.int32(2**30)),
                         axis=1, keepdims=True)
            g = jnp.min(jnp.where(piota == ap, mi_ref[...], jnp.int32(2**30)),
                        axis=1, keepdims=True)
            mv_ref[...] = jnp.where(piota == ap, BIG, vals)
            return jnp.where(lane16 == t, g, gc)

        gc_out[...] = jax.lax.fori_loop(
            0, NSEL, _merge, jnp.zeros((QBLK, NSEL), jnp.int32))


def _k1(queries, keys_padded):
    return pl.pallas_call(
        _k1_body,
        grid=(Q // QBLK, NKSTEP),
        in_specs=[
            pl.BlockSpec((QBLK, D), lambda qi, ki: (qi, 0)),
            pl.BlockSpec((KBLK, D), lambda qi, ki: (ki, 0)),
        ],
        out_specs=[
            pl.BlockSpec((QBLK, KBLK), lambda qi, ki: (qi, ki)),
            pl.BlockSpec((QBLK, NSEL), lambda qi, ki: (qi, 0)),
            pl.BlockSpec((QBLK, 1), lambda qi, ki: (qi, 0)),
        ],
        out_shape=[
            jax.ShapeDtypeStruct((Q, KPAD), jnp.float32),
            jax.ShapeDtypeStruct((Q, NSEL), jnp.int32),
            jax.ShapeDtypeStruct((Q, 1), jnp.float32),
        ],
        scratch_shapes=[pltpu.VMEM((QBLK, NGRP), jnp.float32),
                        pltpu.VMEM((QBLK, MERGEW), jnp.float32),
                        pltpu.VMEM((QBLK, MERGEW), jnp.int32)],
        compiler_params=pltpu.CompilerParams(
            dimension_semantics=("arbitrary", "arbitrary")),
    )(queries, keys_padded)


def _k4_body(a_ref, w_ref, o_ref):
    o_ref[...] = jnp.maximum(
        jnp.dot(a_ref[...].astype(jnp.bfloat16),
                w_ref[...].astype(jnp.bfloat16),
                preferred_element_type=jnp.float32),
        0.0)


def _k4(agg, W):
    return pl.pallas_call(
        _k4_body,
        out_shape=jax.ShapeDtypeStruct((Q, D), jnp.float32),
    )(agg, W)


@jax.jit
def kernel(queries, keys, W):
    keys_padded = jnp.pad(keys, ((0, KPAD - K), (0, 0)),
                          constant_values=1.0e17)
    scores, gcols, q2 = _k1(queries, keys_padded)
    ki_g = gcols // GCOLS
    c_g = gcols % GCOLS
    cand_idx = (ki_g[:, :, None] * KBLK + c_g[:, :, None] +
                jnp.arange(GRP)[None, None, :] * GCOLS).reshape(Q, NSEL * GRP)
    fidx = (jnp.arange(Q)[:, None] * KPAD + cand_idx).reshape(-1)
    cand = jnp.take(scores.reshape(Q * KPAD), fidx).reshape(Q, NSEL * GRP)
    neg, sel = jax.lax.top_k(-cand, TOP_K)
    idx = jnp.take_along_axis(cand_idx, sel, axis=1)
    d = jnp.maximum(-neg + q2, 0.0)
    w = 1.0 / (1.0 + d)
    w = w / jnp.sum(w, axis=1, keepdims=True)
    neighbors = jnp.take(keys_padded, idx, axis=0)
    agg = jnp.sum(w[..., None] * neighbors, axis=1)
    return _k4(agg, W)


# speedup vs baseline: 3.3357x; 1.0239x over previous
import functools
import jax
import jax.numpy as jnp
from jax.experimental import pallas as pl
from jax.experimental.pallas import tpu as pltpu

TOP_K = 16
Q, K, D = 1024, 100000, 128
KPAD = 102400
GRP = 32
NGRP = KPAD // GRP
QBLK = 512
KBLK = 4096
NKSTEP = KPAD // KBLK
GCOLS = KBLK // GRP
NSEL = 16
BIG = float(3.0e38)


NCHUNK = NGRP // GCOLS
MERGEW = 512


def _k1_body(q_ref, k_ref, s_out, gc_out, q2_out, gm_ref, mv_ref, mi_ref):
    ki = pl.program_id(1)
    qb = q_ref[...]
    kb = k_ref[...]
    cross = jax.lax.dot_general(
        (-2.0 * qb).astype(jnp.bfloat16), kb.astype(jnp.bfloat16),
        (((1,), (1,)), ((), ())),
        preferred_element_type=jnp.float32)
    k2b = jax.lax.dot_general(
        jnp.ones((QBLK, D), jnp.float32), kb * kb,
        (((1,), (1,)), ((), ())),
        precision=jax.lax.Precision.HIGHEST,
        preferred_element_type=jnp.float32)
    s_out[...] = cross + k2b

    def _gm_iter(i, m):
        return jnp.minimum(m, s_out[:, pl.ds(i * GCOLS, GCOLS)])
    gm_ref[:, pl.ds(ki * GCOLS, GCOLS)] = jax.lax.fori_loop(
        1, GRP, _gm_iter, s_out[:, 0:GCOLS])

    q2_out[...] = jnp.sum(qb * qb, axis=1, keepdims=True)
    gc_out[...] = jnp.zeros((QBLK, NSEL), jnp.int32)

    @pl.when(ki == NKSTEP - 1)
    def _():
        lane16 = jax.lax.broadcasted_iota(jnp.int32, (QBLK, NSEL), 1)
        ciota = jax.lax.broadcasted_iota(jnp.int32, (QBLK, GCOLS), 1)

        all_v, all_i = [], []
        for j in range(NCHUNK):
            gids = ciota + j * GCOLS
            sl = slice(j * GCOLS, (j + 1) * GCOLS)

            def _sel(t, carry, sl=sl, gids=gids):
                vcol, icol = carry
                vals = gm_ref[:, sl]
                m = jnp.min(vals, axis=1, keepdims=True)
                am = jnp.min(jnp.where(vals == m, gids, jnp.int32(2**30)),
                             axis=1, keepdims=True)
                gm_ref[:, sl] = jnp.where(gids == am, BIG, vals)
                return (jnp.where(lane16 == t, m, vcol),
                        jnp.where(lane16 == t, am, icol))

            vcol, icol = jax.lax.fori_loop(
                0, NSEL, _sel,
                (jnp.full((QBLK, NSEL), BIG, jnp.float32),
                 jnp.zeros((QBLK, NSEL), jnp.int32)))
            all_v.append(vcol)
            all_i.append(icol)
        padw = MERGEW - NCHUNK * NSEL
        mv_ref[...] = jnp.concatenate(
            all_v + [jnp.full((QBLK, padw), BIG, jnp.float32)], axis=1)
        mi_ref[...] = jnp.concatenate(
            all_i + [jnp.zeros((QBLK, padw), jnp.int32)], axis=1)

        piota = jax.lax.broadcasted_iota(jnp.int32, (QBLK, MERGEW), 1)

        def _merge(t, gc):
            vals = mv_ref[...]
            m = jnp.min(vals, axis=1, keepdims=True)
            ap = jnp.min(jnp.where(vals == m, piota, jnp.int32(2**30)),
                         axis=1, keepdims=True)
            g = jnp.min(jnp.where(piota == ap, mi_ref[...], jnp.int32(2**30)),
                        axis=1, keepdims=True)
            mv_ref[...] = jnp.where(piota == ap, BIG, vals)
            return jnp.where(lane16 == t, g, gc)

        gc_out[...] = jax.lax.fori_loop(
            0, NSEL, _merge, jnp.zeros((QBLK, NSEL), jnp.int32))


def _k1(queries, keys_padded):
    return pl.pallas_call(
        _k1_body,
        grid=(Q // QBLK, NKSTEP),
        in_specs=[
            pl.BlockSpec((QBLK, D), lambda qi, ki: (qi, 0)),
            pl.BlockSpec((KBLK, D), lambda qi, ki: (ki, 0)),
        ],
        out_specs=[
            pl.BlockSpec((QBLK, KBLK), lambda qi, ki: (qi, ki)),
            pl.BlockSpec((QBLK, NSEL), lambda qi, ki: (qi, 0)),
            pl.BlockSpec((QBLK, 1), lambda qi, ki: (qi, 0)),
        ],
        out_shape=[
            jax.ShapeDtypeStruct((Q, KPAD), jnp.float32),
            jax.ShapeDtypeStruct((Q, NSEL), jnp.int32),
            jax.ShapeDtypeStruct((Q, 1), jnp.float32),
        ],
        scratch_shapes=[pltpu.VMEM((QBLK, NGRP), jnp.float32),
                        pltpu.VMEM((QBLK, MERGEW), jnp.float32),
                        pltpu.VMEM((QBLK, MERGEW), jnp.int32)],
        compiler_params=pltpu.CompilerParams(
            dimension_semantics=("arbitrary", "arbitrary")),
    )(queries, keys_padded)


CANDW = NSEL * GRP


def _k2_body(cand_ref, cidx_ref, q2_ref, w_out, idx_out, sc_ref):
    sc_ref[...] = cand_ref[...]
    lane16 = jax.lax.broadcasted_iota(jnp.int32, (Q, NSEL), 1)
    piota = jax.lax.broadcasted_iota(jnp.int32, (Q, CANDW), 1)

    def _sel(t, carry):
        vval, vidx = carry
        vals = sc_ref[...]
        m = jnp.min(vals, axis=1, keepdims=True)
        ap = jnp.min(jnp.where(vals == m, piota, jnp.int32(2**30)),
                     axis=1, keepdims=True)
        g = jnp.min(jnp.where(piota == ap, cidx_ref[...], jnp.int32(2**30)),
                    axis=1, keepdims=True)
        sc_ref[...] = jnp.where(piota == ap, BIG, vals)
        return (jnp.where(lane16 == t, m, vval),
                jnp.where(lane16 == t, g, vidx))

    vval, vidx = jax.lax.fori_loop(
        0, NSEL, _sel,
        (jnp.full((Q, NSEL), BIG, jnp.float32),
         jnp.zeros((Q, NSEL), jnp.int32)))
    d = jnp.maximum(vval + q2_ref[...], 0.0)
    w = 1.0 / (1.0 + d)
    w_out[...] = w / jnp.sum(w, axis=1, keepdims=True)
    idx_out[...] = vidx


def _k2(cand, cidx, q2):
    return pl.pallas_call(
        _k2_body,
        out_shape=[jax.ShapeDtypeStruct((Q, NSEL), jnp.float32),
                   jax.ShapeDtypeStruct((Q, NSEL), jnp.int32)],
        scratch_shapes=[pltpu.VMEM((Q, CANDW), jnp.float32)],
    )(cand, cidx, q2)


def _k3_body(w_ref, nb_ref, o_ref):
    acc = w_ref[:, 0:1] * nb_ref[:, 0, :]
    for g in range(1, NSEL):
        acc = acc + w_ref[:, g:g + 1] * nb_ref[:, g, :]
    o_ref[...] = acc


def _k3(wn, neighbors):
    return pl.pallas_call(
        _k3_body,
        out_shape=jax.ShapeDtypeStruct((Q, D), jnp.float32),
    )(wn, neighbors)


def _k4_body(a_ref, w_ref, o_ref):
    o_ref[...] = jnp.maximum(
        jnp.dot(a_ref[...].astype(jnp.bfloat16),
                w_ref[...].astype(jnp.bfloat16),
                preferred_element_type=jnp.float32),
        0.0)


def _k4(agg, W):
    return pl.pallas_call(
        _k4_body,
        out_shape=jax.ShapeDtypeStruct((Q, D), jnp.float32),
    )(agg, W)


@jax.jit
def kernel(queries, keys, W):
    keys_padded = jnp.pad(keys, ((0, KPAD - K), (0, 0)),
                          constant_values=1.0e17)
    scores, gcols, q2 = _k1(queries, keys_padded)
    ki_g = gcols // GCOLS
    c_g = gcols % GCOLS
    cand_idx = (ki_g[:, :, None] * KBLK + c_g[:, :, None] +
                jnp.arange(GRP)[None, None, :] * GCOLS).reshape(Q, NSEL * GRP)
    fidx = (jnp.arange(Q)[:, None] * KPAD + cand_idx).reshape(-1)
    cand = jnp.take(scores.reshape(Q * KPAD), fidx).reshape(Q, NSEL * GRP)
    wn, idx = _k2(cand, cand_idx, q2)
    neighbors = jnp.take(keys_padded, idx.reshape(-1),
                         axis=0).reshape(Q, NSEL, D)
    agg = _k3(wn, neighbors)
    return _k4(agg, W)
